# baseline (speedup 1.0000x reference)
import functools

import jax
import jax.numpy as jnp
from jax import lax
from jax.experimental import pallas as pl
from jax.experimental.pallas import tpu as pltpu
from jax.experimental.pallas import tpu_sc as plsc

BATCH = 16384
HIDDEN = 64
NC = 2
NS = 16
L = 16
NW = NC * NS
BPW = BATCH // NW
CH = 128
NCH = BPW // CH
GPC = CH // L
NCHUNK = HIDDEN // L


def _neg_sqrt(x):
    i = lax.bitcast_convert_type(x, jnp.int32)
    y = lax.bitcast_convert_type(jnp.int32(0x5F3759DF) - (i >> 1), jnp.float32)
    for _ in range(3):
        y = y * (1.5 - 0.5 * x * y * y)
    return -(x * y)


@functools.cache
def _build_sc_kernel():
  mesh = plsc.VectorSubcoreMesh(
      core_axis_name="c", subcore_axis_name="s", num_cores=NC, num_subcores=NS
  )

  @functools.partial(
      pl.kernel,
      out_type=jax.ShapeDtypeStruct((BATCH,), jnp.float32),
      mesh=mesh,
      compiler_params=pltpu.CompilerParams(use_tc_tiling_on_sc=False),
      scratch_types=[
        pltpu.VMEM((CH,), jnp.int32),
        pltpu.VMEM((CH,), jnp.int32),
        pltpu.VMEM((CH,), jnp.int32),
        pltpu.VMEM((CH, HIDDEN), jnp.float32),
        pltpu.VMEM((CH, HIDDEN), jnp.float32),
        pltpu.VMEM((CH, HIDDEN), jnp.float32),
        pltpu.VMEM((CH,), jnp.float32),
        pltpu.SemaphoreType.DMA,
        pltpu.SemaphoreType.DMA,
        pltpu.SemaphoreType.DMA,
      ],
  )
  def _sc_kernel(head_hbm, rel_hbm, tail_hbm, node_hbm, relemb_hbm, out_hbm,
                 hidx, ridx, tidx, hrows, rrows, trows, osc, s1, s2, s3):
      wid = lax.axis_index("s") * NC + lax.axis_index("c")
      base = pl.multiple_of(wid * BPW, BPW)
      lane = lax.iota(jnp.int32, L)

      for c in range(NCH):
          cbase = pl.multiple_of(base + c * CH, CH)
          pltpu.sync_copy(head_hbm.at[pl.ds(cbase, CH)], hidx)
          pltpu.sync_copy(rel_hbm.at[pl.ds(cbase, CH)], ridx)
          pltpu.sync_copy(tail_hbm.at[pl.ds(cbase, CH)], tidx)
          hcp = pltpu.async_copy(node_hbm.at[hidx], hrows, s1)
          rcp = pltpu.async_copy(relemb_hbm.at[ridx], rrows, s2)
          tcp = pltpu.async_copy(node_hbm.at[tidx], trows, s3)
          hcp.wait()
          rcp.wait()
          tcp.wait()

          def group_body(g, carry):
              out_vec = jnp.zeros((L,), jnp.float32)
              for j in range(L):
                  row = g * L + j
                  acc = jnp.zeros((L,), jnp.float32)
                  for cc in range(NCHUNK):
                      hv = hrows[row, pl.ds(cc * L, L)]
                      rv = rrows[row, pl.ds(cc * L, L)]
                      tv = trows[row, pl.ds(cc * L, L)]
                      d = (hv + rv) - tv
                      acc = acc + d * d
                  for sh in (8, 4, 2, 1):
                      acc = acc + acc.at[lane ^ sh].get(
                          mode="promise_in_bounds")
                  out_vec = jnp.where(lane == j, acc, out_vec)
              osc[pl.ds(g * L, L)] = _neg_sqrt(out_vec + 1e-12)
              return carry

          lax.fori_loop(0, GPC, group_body, 0)
          pltpu.sync_copy(osc, out_hbm.at[pl.ds(cbase, CH)])

  return _sc_kernel


def kernel(head_index, rel_type, tail_index, node_emb, rel_emb):
    return _build_sc_kernel()(head_index, rel_type, tail_index, node_emb, rel_emb)


# speedup vs baseline: 1.0095x; 1.0095x over previous
import functools

import jax
import jax.numpy as jnp
from jax import lax
from jax.experimental import pallas as pl
from jax.experimental.pallas import tpu as pltpu
from jax.experimental.pallas import tpu_sc as plsc

BATCH = 16384
HIDDEN = 64
NC = 2
NS = 16
L = 16
NW = NC * NS
BPW = BATCH // NW
CH = 128
NCH = BPW // CH
GPC = CH // L


def _neg_sqrt(x):
    i = lax.bitcast_convert_type(x, jnp.int32)
    y = lax.bitcast_convert_type(jnp.int32(0x5F3759DF) - (i >> 1), jnp.float32)
    for _ in range(3):
        y = y * (1.5 - 0.5 * x * y * y)
    return -(x * y)


def _mrg(lane, s, a, b):
    pa = a.at[lane ^ s].get(mode="promise_in_bounds")
    pb = b.at[lane ^ s].get(mode="promise_in_bounds")
    return jnp.where((lane & s) == 0, a + pa, b + pb)


@functools.cache
def _build_sc_kernel():
  mesh = plsc.VectorSubcoreMesh(
      core_axis_name="c", subcore_axis_name="s", num_cores=NC, num_subcores=NS
  )

  @functools.partial(
      pl.kernel,
      out_type=jax.ShapeDtypeStruct((BATCH,), jnp.float32),
      mesh=mesh,
      compiler_params=pltpu.CompilerParams(use_tc_tiling_on_sc=False),
      scratch_types=[
          pltpu.VMEM((CH,), jnp.int32), pltpu.VMEM((CH,), jnp.int32),
          pltpu.VMEM((CH,), jnp.int32), pltpu.VMEM((CH,), jnp.int32),
          pltpu.VMEM((CH,), jnp.int32), pltpu.VMEM((CH,), jnp.int32),
          pltpu.VMEM((CH, HIDDEN), jnp.float32),
          pltpu.VMEM((CH, HIDDEN), jnp.float32),
          pltpu.VMEM((CH, HIDDEN), jnp.float32),
          pltpu.VMEM((CH, HIDDEN), jnp.float32),
          pltpu.VMEM((CH, HIDDEN), jnp.float32),
          pltpu.VMEM((CH, HIDDEN), jnp.float32),
          pltpu.VMEM((CH,), jnp.float32),
          pltpu.VMEM((CH * L // 2,), jnp.float32),
          pltpu.SemaphoreType.DMA,
          pltpu.SemaphoreType.DMA,
      ],
  )
  def _sc_kernel(head_hbm, rel_hbm, tail_hbm, node_hbm, relemb_hbm, out_hbm,
                 hi0, hi1, ri0, ri1, ti0, ti1,
                 hb0, hb1, rb0, rb1, tb0, tb1, osc, pbuf, s0, s1):
      wid = lax.axis_index("s") * NC + lax.axis_index("c")
      base = pl.multiple_of(wid * BPW, BPW)
      lane = lax.iota(jnp.int32, L)
      hidx, ridx, tidx = (hi0, hi1), (ri0, ri1), (ti0, ti1)
      hbuf, rbuf, tbuf = (hb0, hb1), (rb0, rb1), (tb0, tb1)
      sems = (s0, s1)

      def load_idx(c, slot):
          cb = pl.multiple_of(base + c * CH, CH)
          pltpu.sync_copy(head_hbm.at[pl.ds(cb, CH)], hidx[slot])
          pltpu.sync_copy(rel_hbm.at[pl.ds(cb, CH)], ridx[slot])
          pltpu.sync_copy(tail_hbm.at[pl.ds(cb, CH)], tidx[slot])

      def start_gather(slot):
          sem = sems[slot]
          return (
              pltpu.async_copy(node_hbm.at[hidx[slot]], hbuf[slot], sem),
              pltpu.async_copy(relemb_hbm.at[ridx[slot]], rbuf[slot], sem),
              pltpu.async_copy(node_hbm.at[tidx[slot]], tbuf[slot], sem),
          )

      load_idx(0, 0)
      inflight = {0: start_gather(0)}
      load_idx(1, 1)
      inflight[1] = start_gather(1)

      for c in range(NCH):
          slot = c & 1
          for cp in inflight[slot]:
              cp.wait()
          hb, rb, tb = hbuf[slot], rbuf[slot], tbuf[slot]

          def pair_body(p, carry, hb=hb, rb=rb, tb=tb):
              g = p >> 3
              r = p & 7
              rowa = (g << 4) + r

              def rowacc(row):
                  acc = None
                  for cc in range(HIDDEN // L):
                      hv = hb[row, pl.ds(cc * L, L)]
                      rv = rb[row, pl.ds(cc * L, L)]
                      tv = tb[row, pl.ds(cc * L, L)]
                      dd = (hv + rv) - tv
                      sq = dd * dd
                      acc = sq if acc is None else acc + sq
                  return acc

              pbuf[pl.ds(p * L, L)] = _mrg(lane, 8, rowacc(rowa),
                                           rowacc(rowa + 8))
              return carry

          def group_body(g, carry):
              b8 = g * 8
              q = [pbuf[pl.ds((b8 + r) * L, L)] for r in range(8)]
              t = [_mrg(lane, 4, q[r], q[r + 4]) for r in range(4)]
              e0 = _mrg(lane, 2, t[0], t[2])
              e1 = _mrg(lane, 2, t[1], t[3])
              tot = _mrg(lane, 1, e0, e1)
              osc[pl.ds(g * L, L)] = _neg_sqrt(tot + 1e-12)
              return carry

          lax.fori_loop(0, CH // 2, pair_body, 0)
          lax.fori_loop(0, GPC, group_body, 0)
          cb = pl.multiple_of(base + c * CH, CH)
          pltpu.sync_copy(osc, out_hbm.at[pl.ds(cb, CH)])
          if c + 2 < NCH:
              load_idx(c + 2, slot)
              inflight[slot] = start_gather(slot)

  return _sc_kernel


def kernel(head_index, rel_type, tail_index, node_emb, rel_emb):
    return _build_sc_kernel()(head_index, rel_type, tail_index, node_emb, rel_emb)
